# Initial kernel scaffold; baseline (speedup 1.0000x reference)
import functools

import jax
import jax.numpy as jnp
from jax import lax
from jax.experimental import pallas as pl
from jax.experimental.pallas import tpu as pltpu
from jax.experimental.pallas import tpu_sc as plsc

_N = 10000
_NS = 5000
_R, _C = 8, 1280
_PAD = _R * _C


def _fps_kernel(px_ref, py_ref, pz_ref, d0_ref, sel_ref):
    px = px_ref[...]
    py = py_ref[...]
    pz = pz_ref[...]
    flat = (lax.broadcasted_iota(jnp.int32, (_R, _C), 0) * _C
            + lax.broadcasted_iota(jnp.int32, (_R, _C), 1))

    sel_ref[0:1, :] = jnp.zeros((1, 1), jnp.int32)

    def pick(mask, arr):
        return jnp.sum(jnp.where(mask, arr, 0.0))

    m0 = flat == 0
    px0, py0, pz0 = pick(m0, px), pick(m0, py), pick(m0, pz)

    def body(i, carry):
        dists, pxs, pys, pzs = carry
        dx = px - pxs
        dy = py - pys
        dz = pz - pzs
        d = (dx * dx + dy * dy) + dz * dz
        dists = jnp.minimum(dists, d)
        mx = jnp.max(dists)
        nxt = jnp.min(jnp.where(dists == mx, flat, jnp.int32(_PAD)))
        sel_ref[pl.ds(i, 1), :] = jnp.broadcast_to(nxt, (1, 1))
        msel = flat == nxt
        return dists, pick(msel, px), pick(msel, py), pick(msel, pz)

    lax.fori_loop(1, _NS, body, (d0_ref[...], px0, py0, pz0))


def _run_fps(pos):
    posp = jnp.pad(pos, ((0, _PAD - _N), (0, 0)))
    px = posp[:, 0].reshape(_R, _C)
    py = posp[:, 1].reshape(_R, _C)
    pz = posp[:, 2].reshape(_R, _C)
    valid = (lax.broadcasted_iota(jnp.int32, (_R, _C), 0) * _C
             + lax.broadcasted_iota(jnp.int32, (_R, _C), 1)) < _N
    d0 = jnp.where(valid, jnp.inf, -jnp.inf).astype(jnp.float32)
    sel = pl.pallas_call(
        _fps_kernel,
        out_shape=jax.ShapeDtypeStruct((_NS, 1), jnp.int32),
    )(px, py, pz, d0)
    return sel.reshape(_NS)


_NC, _NSUB = 2, 16
_NW = _NC * _NSUB
_CHUNK = 80
_WROWS = 2 * _CHUNK


def _sc_gather_body(x_hbm, pos_hbm, bat_hbm, idx_hbm,
                    xo_hbm, po_hbm, bo_hbm,
                    idx_v, xb, pb, bb, *sems):
    wid = lax.axis_index("s") * _NC + lax.axis_index("c")
    base = jnp.minimum(wid * _WROWS, _NS - _WROWS)

    for j in range(2):
        pltpu.sync_copy(idx_hbm.at[pl.ds(base + j * _CHUNK, _CHUNK)],
                        idx_v.at[j])

    tabs = (x_hbm, pos_hbm, bat_hbm)
    bufs = (xb, pb, bb)
    outs = (xo_hbm, po_hbm, bo_hbm)
    cps = []
    k = 0
    for j in range(2):
        for t in range(3):
            cps.append(pltpu.async_copy(tabs[t].at[idx_v.at[j]],
                                        bufs[t].at[j], sems[k]))
            k += 1
    k = 0
    for j in range(2):
        for t in range(3):
            cps[k].wait()
            k += 1
            pltpu.sync_copy(bufs[t].at[j],
                            outs[t].at[pl.ds(base + j * _CHUNK, _CHUNK)])


_sc_gather = functools.partial(
    pl.kernel,
    out_type=(
        jax.ShapeDtypeStruct((_NS, 128), jnp.float32),
        jax.ShapeDtypeStruct((_NS, 16), jnp.float32),
        jax.ShapeDtypeStruct((_NS, 16), jnp.int32),
    ),
    mesh=plsc.VectorSubcoreMesh(core_axis_name="c", subcore_axis_name="s"),
    scratch_types=[
        pltpu.VMEM((2, _CHUNK), jnp.int32),
        pltpu.VMEM((2, _CHUNK, 128), jnp.float32),
        pltpu.VMEM((2, _CHUNK, 16), jnp.float32),
        pltpu.VMEM((2, _CHUNK, 16), jnp.int32),
    ] + [pltpu.SemaphoreType.DMA] * 6,
)(_sc_gather_body)


@jax.jit
def kernel(x, pos, batch):
    idx = _run_fps(pos)
    pos16 = jnp.pad(pos, ((0, 0), (0, 13)))
    bat16 = jnp.pad(batch[:, None], ((0, 0), (0, 15)))
    xo, po, bo = _sc_gather(x, pos16, bat16, idx)
    return xo, po[:, :3], bo[:, 0]


# speedup vs baseline: 11.3851x; 11.3851x over previous
import functools

import jax
import jax.numpy as jnp
from jax import lax
from jax.experimental import pallas as pl
from jax.experimental.pallas import tpu as pltpu
from jax.experimental.pallas import tpu_sc as plsc

_N = 10000
_NS = 5000
_R, _C = 8, 1280
_PAD = _R * _C


def _fps_kernel(px_ref, py_ref, pz_ref, d0_ref, sel_ref):
    px = px_ref[...]
    py = py_ref[...]
    pz = pz_ref[...]
    flat = (lax.broadcasted_iota(jnp.int32, (_R, _C), 0) * _C
            + lax.broadcasted_iota(jnp.int32, (_R, _C), 1))

    sel_ref[0:1, :] = jnp.zeros((1, 1), jnp.int32)

    def pick(mask, arr):
        return jnp.sum(jnp.where(mask, arr, 0.0))

    m0 = flat == 0
    px0, py0, pz0 = pick(m0, px), pick(m0, py), pick(m0, pz)

    def body(i, carry):
        dists, pxs, pys, pzs = carry
        dx = px - pxs
        dy = py - pys
        dz = pz - pzs
        d = (dx * dx + dy * dy) + dz * dz
        dists = jnp.minimum(dists, d)
        mx = jnp.max(dists)
        nxt = jnp.min(jnp.where(dists == mx, flat, jnp.int32(_PAD)))
        sel_ref[pl.ds(i, 1), :] = jnp.broadcast_to(nxt, (1, 1))
        msel = flat == nxt
        return dists, pick(msel, px), pick(msel, py), pick(msel, pz)

    lax.fori_loop(1, _NS, body, (d0_ref[...], px0, py0, pz0))


def _run_fps(pos):
    posp = jnp.pad(pos, ((0, _PAD - _N), (0, 0)))
    px = posp[:, 0].reshape(_R, _C)
    py = posp[:, 1].reshape(_R, _C)
    pz = posp[:, 2].reshape(_R, _C)
    valid = (lax.broadcasted_iota(jnp.int32, (_R, _C), 0) * _C
             + lax.broadcasted_iota(jnp.int32, (_R, _C), 1)) < _N
    d0 = jnp.where(valid, jnp.inf, -jnp.inf).astype(jnp.float32)
    sel = pl.pallas_call(
        _fps_kernel,
        out_shape=jax.ShapeDtypeStruct((_NS, 1), jnp.int32),
    )(px, py, pz, d0)
    return sel.reshape(_NS)


_NC, _NSUB = 2, 16
_NW = _NC * _NSUB
_CHUNK = 80
_WROWS = 2 * _CHUNK


def _sc_gather_body(x_hbm, pos_hbm, bat_hbm, idx_hbm,
                    xo_hbm, po_hbm, bo_hbm,
                    idx_v, xb, pb, bb, *sems):
    wid = lax.axis_index("s") * _NC + lax.axis_index("c")
    base = jnp.minimum(wid * _WROWS, _NS - _WROWS)

    for j in range(2):
        pltpu.sync_copy(idx_hbm.at[pl.ds(base + j * _CHUNK, _CHUNK)],
                        idx_v.at[j])

    tabs = (x_hbm, pos_hbm, bat_hbm)
    bufs = (xb, pb, bb)
    outs = (xo_hbm, po_hbm, bo_hbm)
    cps = []
    k = 0
    for j in range(2):
        for t in range(3):
            cps.append(pltpu.async_copy(tabs[t].at[idx_v.at[j]],
                                        bufs[t].at[j], sems[k]))
            k += 1
    k = 0
    for j in range(2):
        for t in range(3):
            cps[k].wait()
            k += 1
            pltpu.sync_copy(bufs[t].at[j],
                            outs[t].at[pl.ds(base + j * _CHUNK, _CHUNK)])


@functools.lru_cache(maxsize=1)
def _sc_gather():
    return pl.kernel(
        _sc_gather_body,
        out_type=(
            jax.ShapeDtypeStruct((_NS, 128), jnp.float32),
            jax.ShapeDtypeStruct((_NS, 16), jnp.float32),
            jax.ShapeDtypeStruct((_NS, 16), jnp.int32),
        ),
        mesh=plsc.VectorSubcoreMesh(core_axis_name="c", subcore_axis_name="s"),
        scratch_types=[
            pltpu.VMEM((2, _CHUNK), jnp.int32),
            pltpu.VMEM((2, _CHUNK, 128), jnp.float32),
            pltpu.VMEM((2, _CHUNK, 16), jnp.float32),
            pltpu.VMEM((2, _CHUNK, 16), jnp.int32),
        ] + [pltpu.SemaphoreType.DMA] * 6,
        compiler_params=pltpu.CompilerParams(use_tc_tiling_on_sc=False),
    )


@jax.jit
def kernel(x, pos, batch):
    idx = _run_fps(pos)
    pos16 = jnp.pad(pos, ((0, 0), (0, 13)))
    bat16 = jnp.pad(batch[:, None], ((0, 0), (0, 15)))
    xo, po, bo = _sc_gather()(x, pos16, bat16, idx)
    return xo, po[:, :3], bo[:, 0]


# speedup vs baseline: 22.9942x; 2.0197x over previous
import functools

import jax
import jax.numpy as jnp
from jax import lax
from jax.experimental import pallas as pl
from jax.experimental.pallas import tpu as pltpu
from jax.experimental.pallas import tpu_sc as plsc

_N = 10000
_NS = 5000
_R, _C = 8, 1280
_PAD = _R * _C


_BIG = 3.0e38
_RR = _PAD // 128
_G = _RR // 8


def _fps_kernel(px_ref, py_ref, pz_ref, d0_ref,
                cxr_ref, cyr_ref, czr_ref, sel_ref):
    px = px_ref[...]
    py = py_ref[...]
    pz = pz_ref[...]
    flat_f = (lax.broadcasted_iota(jnp.int32, (_RR, 128), 0) * 128
              + lax.broadcasted_iota(jnp.int32, (_RR, 128), 1)
              ).astype(jnp.float32)

    sel_ref[0:1, :] = jnp.zeros((1, 1), jnp.int32)

    def body(i, carry):
        dists, cx, cy, cz = carry
        dx = px - cx
        dy = py - cy
        dz = pz - cz
        d = (dx * dx + dy * dy) + dz * dz
        dists = jnp.minimum(dists, d)
        items = [(dists[g * 8:(g + 1) * 8, :], flat_f[g * 8:(g + 1) * 8, :])
                 for g in range(_G)]
        while len(items) > 1:
            nitems = []
            for k in range(0, len(items) - 1, 2):
                (va, ia), (vb, ib) = items[k], items[k + 1]
                take = vb > va
                nitems.append((jnp.where(take, vb, va),
                               jnp.where(take, ib, ia)))
            if len(items) % 2:
                nitems.append(items[-1])
            items = nitems
        v8, i8 = items[0]
        t = v8
        for sh in (4, 2, 1):
            t = jnp.maximum(t, pltpu.roll(t, sh, 0))
        mx = jnp.max(t, axis=1, keepdims=True)
        nxt_f = jnp.min(jnp.where(v8 == mx, i8, _BIG))
        nxt = nxt_f.astype(jnp.int32)
        sel_ref[pl.ds(i, 1), :] = jnp.broadcast_to(nxt, (1, 1))
        cx = cxr_ref[pl.ds(nxt, 1), :]
        cy = cyr_ref[pl.ds(nxt, 1), :]
        cz = czr_ref[pl.ds(nxt, 1), :]
        return dists, cx, cy, cz

    c0 = (cxr_ref[0:1, :], cyr_ref[0:1, :], czr_ref[0:1, :])
    lax.fori_loop(1, _NS, body, (d0_ref[...],) + c0)


def _run_fps(pos):
    posp = jnp.pad(pos, ((0, _PAD - _N), (0, 0)))
    px = posp[:, 0].reshape(_RR, 128)
    py = posp[:, 1].reshape(_RR, 128)
    pz = posp[:, 2].reshape(_RR, 128)
    pxr = jnp.broadcast_to(posp[:, 0:1], (_PAD, 128))
    pyr = jnp.broadcast_to(posp[:, 1:2], (_PAD, 128))
    pzr = jnp.broadcast_to(posp[:, 2:3], (_PAD, 128))
    valid = (lax.broadcasted_iota(jnp.int32, (_RR, 128), 0) * 128
             + lax.broadcasted_iota(jnp.int32, (_RR, 128), 1)) < _N
    d0 = jnp.where(valid, jnp.inf, -jnp.inf).astype(jnp.float32)
    sel = pl.pallas_call(
        _fps_kernel,
        out_shape=jax.ShapeDtypeStruct((_NS, 1), jnp.int32),
    )(px, py, pz, d0, pxr, pyr, pzr)
    return sel.reshape(_NS)


_NC, _NSUB = 2, 16
_NW = _NC * _NSUB
_CHUNK = 80
_WROWS = 2 * _CHUNK


def _sc_gather_body(x_hbm, pos_hbm, bat_hbm, idx_hbm,
                    xo_hbm, po_hbm, bo_hbm,
                    idx_v, xb, pb, bb, *sems):
    wid = lax.axis_index("s") * _NC + lax.axis_index("c")
    base = jnp.minimum(wid * _WROWS, _NS - _WROWS)

    for j in range(2):
        pltpu.sync_copy(idx_hbm.at[pl.ds(base + j * _CHUNK, _CHUNK)],
                        idx_v.at[j])

    tabs = (x_hbm, pos_hbm, bat_hbm)
    bufs = (xb, pb, bb)
    outs = (xo_hbm, po_hbm, bo_hbm)
    cps = []
    k = 0
    for j in range(2):
        for t in range(3):
            cps.append(pltpu.async_copy(tabs[t].at[idx_v.at[j]],
                                        bufs[t].at[j], sems[k]))
            k += 1
    k = 0
    for j in range(2):
        for t in range(3):
            cps[k].wait()
            k += 1
            pltpu.sync_copy(bufs[t].at[j],
                            outs[t].at[pl.ds(base + j * _CHUNK, _CHUNK)])


@functools.lru_cache(maxsize=1)
def _sc_gather():
    return pl.kernel(
        _sc_gather_body,
        out_type=(
            jax.ShapeDtypeStruct((_NS, 128), jnp.float32),
            jax.ShapeDtypeStruct((_NS, 16), jnp.float32),
            jax.ShapeDtypeStruct((_NS, 16), jnp.int32),
        ),
        mesh=plsc.VectorSubcoreMesh(core_axis_name="c", subcore_axis_name="s"),
        scratch_types=[
            pltpu.VMEM((2, _CHUNK), jnp.int32),
            pltpu.VMEM((2, _CHUNK, 128), jnp.float32),
            pltpu.VMEM((2, _CHUNK, 16), jnp.float32),
            pltpu.VMEM((2, _CHUNK, 16), jnp.int32),
        ] + [pltpu.SemaphoreType.DMA] * 6,
        compiler_params=pltpu.CompilerParams(use_tc_tiling_on_sc=False),
    )


@jax.jit
def kernel(x, pos, batch):
    idx = _run_fps(pos)
    pos16 = jnp.pad(pos, ((0, 0), (0, 13)))
    bat16 = jnp.pad(batch[:, None], ((0, 0), (0, 15)))
    xo, po, bo = _sc_gather()(x, pos16, bat16, idx)
    return xo, po[:, :3], bo[:, 0]


# speedup vs baseline: 23.2087x; 1.0093x over previous
import functools

import jax
import jax.numpy as jnp
from jax import lax
from jax.experimental import pallas as pl
from jax.experimental.pallas import tpu as pltpu
from jax.experimental.pallas import tpu_sc as plsc

_N = 10000
_NS = 5000
_R, _C = 8, 1280
_PAD = _R * _C


_BIG = 3.0e38
_RR = _PAD // 128
_G = _RR // 8


def _fps_kernel(px_ref, py_ref, pz_ref, d0_ref,
                cxr_ref, cyr_ref, czr_ref, sel_ref):
    px = px_ref[...]
    py = py_ref[...]
    pz = pz_ref[...]
    flat_f = (lax.broadcasted_iota(jnp.int32, (_RR, 128), 0) * 128
              + lax.broadcasted_iota(jnp.int32, (_RR, 128), 1)
              ).astype(jnp.float32) + 8388608.0

    sel_ref[0:1, :] = jnp.zeros((1, 1), jnp.int32)

    def body(i, carry):
        dists, cx, cy, cz = carry
        dx = px - cx
        dy = py - cy
        dz = pz - cz
        d = (dx * dx + dy * dy) + dz * dz
        dists = jnp.minimum(dists, d)
        items = [(dists[g * 8:(g + 1) * 8, :], flat_f[g * 8:(g + 1) * 8, :])
                 for g in range(_G)]
        while len(items) > 1:
            nitems = []
            for k in range(0, len(items) - 1, 2):
                (va, ia), (vb, ib) = items[k], items[k + 1]
                nitems.append((jnp.maximum(va, vb),
                               jnp.where(vb > va, ib, ia)))
            if len(items) % 2:
                nitems.append(items[-1])
            items = nitems
        v8, i8 = items[0]
        t = v8
        for sh in (4, 2, 1):
            t = jnp.maximum(t, pltpu.roll(t, sh, 0))
        mx = jnp.max(t, axis=1, keepdims=True)
        nxt_f = jnp.min(jnp.where(v8 == mx, i8, _BIG))
        nxt = lax.bitcast_convert_type(nxt_f, jnp.int32) & 0x7FFFFF
        sel_ref[pl.ds(i, 1), :] = jnp.broadcast_to(nxt, (1, 1))
        cx = cxr_ref[pl.ds(nxt, 1), :]
        cy = cyr_ref[pl.ds(nxt, 1), :]
        cz = czr_ref[pl.ds(nxt, 1), :]
        return dists, cx, cy, cz

    c0 = (cxr_ref[0:1, :], cyr_ref[0:1, :], czr_ref[0:1, :])
    lax.fori_loop(1, _NS, body, (d0_ref[...],) + c0)


def _run_fps(pos):
    posp = jnp.pad(pos, ((0, _PAD - _N), (0, 0)))
    px = posp[:, 0].reshape(_RR, 128)
    py = posp[:, 1].reshape(_RR, 128)
    pz = posp[:, 2].reshape(_RR, 128)
    pxr = jnp.broadcast_to(posp[:, 0:1], (_PAD, 128))
    pyr = jnp.broadcast_to(posp[:, 1:2], (_PAD, 128))
    pzr = jnp.broadcast_to(posp[:, 2:3], (_PAD, 128))
    valid = (lax.broadcasted_iota(jnp.int32, (_RR, 128), 0) * 128
             + lax.broadcasted_iota(jnp.int32, (_RR, 128), 1)) < _N
    d0 = jnp.where(valid, jnp.inf, -jnp.inf).astype(jnp.float32)
    sel = pl.pallas_call(
        _fps_kernel,
        out_shape=jax.ShapeDtypeStruct((_NS, 1), jnp.int32),
    )(px, py, pz, d0, pxr, pyr, pzr)
    return sel.reshape(_NS)


_NC, _NSUB = 2, 16
_NW = _NC * _NSUB
_CHUNK = 80
_WROWS = 2 * _CHUNK


def _sc_gather_body(x_hbm, pos_hbm, bat_hbm, idx_hbm,
                    xo_hbm, po_hbm, bo_hbm,
                    idx_v, xb, pb, bb, *sems):
    wid = lax.axis_index("s") * _NC + lax.axis_index("c")
    base = jnp.minimum(wid * _WROWS, _NS - _WROWS)

    for j in range(2):
        pltpu.sync_copy(idx_hbm.at[pl.ds(base + j * _CHUNK, _CHUNK)],
                        idx_v.at[j])

    tabs = (x_hbm, pos_hbm, bat_hbm)
    bufs = (xb, pb, bb)
    outs = (xo_hbm, po_hbm, bo_hbm)
    cps = []
    k = 0
    for j in range(2):
        for t in range(3):
            cps.append(pltpu.async_copy(tabs[t].at[idx_v.at[j]],
                                        bufs[t].at[j], sems[k]))
            k += 1
    k = 0
    for j in range(2):
        for t in range(3):
            cps[k].wait()
            k += 1
            pltpu.sync_copy(bufs[t].at[j],
                            outs[t].at[pl.ds(base + j * _CHUNK, _CHUNK)])


@functools.lru_cache(maxsize=1)
def _sc_gather():
    return pl.kernel(
        _sc_gather_body,
        out_type=(
            jax.ShapeDtypeStruct((_NS, 128), jnp.float32),
            jax.ShapeDtypeStruct((_NS, 16), jnp.float32),
            jax.ShapeDtypeStruct((_NS, 16), jnp.int32),
        ),
        mesh=plsc.VectorSubcoreMesh(core_axis_name="c", subcore_axis_name="s"),
        scratch_types=[
            pltpu.VMEM((2, _CHUNK), jnp.int32),
            pltpu.VMEM((2, _CHUNK, 128), jnp.float32),
            pltpu.VMEM((2, _CHUNK, 16), jnp.float32),
            pltpu.VMEM((2, _CHUNK, 16), jnp.int32),
        ] + [pltpu.SemaphoreType.DMA] * 6,
        compiler_params=pltpu.CompilerParams(use_tc_tiling_on_sc=False),
    )


@jax.jit
def kernel(x, pos, batch):
    idx = _run_fps(pos)
    pos16 = jnp.pad(pos, ((0, 0), (0, 13)))
    bat16 = jnp.pad(batch[:, None], ((0, 0), (0, 15)))
    xo, po, bo = _sc_gather()(x, pos16, bat16, idx)
    return xo, po[:, :3], bo[:, 0]
